# baseline (speedup 1.0000x reference)
import functools

import jax
import jax.numpy as jnp
from jax import lax
from jax.experimental import pallas as pl
from jax.experimental.pallas import tpu as pltpu
from jax.experimental.pallas import tpu_sc as plsc

N_NODES = 10000
N_EDGES = 320000
D_FEAT = 128

NUM_CORES = 2
NUM_TILES = 16
CHUNK = 128
D_HALF = D_FEAT // NUM_CORES
NBUF = 4

CHUNKS_PER_TILE = 160
E_PAD = CHUNKS_PER_TILE * NUM_TILES * CHUNK
ZERO_ROWS = 632
ROWS_PAD = ZERO_ROWS * NUM_TILES
OUT_ROWS_LAST = N_NODES - ZERO_ROWS * (NUM_TILES - 1)


def _sc_kernel(x_lo_hbm, x_hi_hbm, src_hbm, dst_hbm, z_hbm, out_hbm,
               acc, src_v, dst_v, rows, gsem, ssem):
    c = lax.axis_index("c")
    s = lax.axis_index("s")

    pltpu.sync_copy(z_hbm.at[pl.ds(s * ZERO_ROWS, ZERO_ROWS)],
                    acc.at[pl.ds(s * ZERO_ROWS, ZERO_ROWS)])

    pltpu.sync_copy(src_hbm.at[s], src_v)
    pltpu.sync_copy(dst_hbm.at[s], dst_v)
    plsc.subcore_barrier()

    def gather(j, b):
        @pl.when(c == 0)
        def _():
            pltpu.async_copy(x_lo_hbm.at[src_v.at[j]], rows[b], gsem[b])

        @pl.when(c == 1)
        def _():
            pltpu.async_copy(x_hi_hbm.at[src_v.at[j]], rows[b], gsem[b])

    def gather_wait(b):
        pltpu.make_async_copy(x_lo_hbm.at[pl.ds(0, CHUNK)], rows[b],
                              gsem[b]).wait()

    def scatter_wait(b):
        pltpu.make_async_copy(rows[b], acc.at[pl.ds(0, CHUNK)],
                              ssem[b]).wait()

    gather(0, 0)
    gather(1, 1)

    def chunk_body(i, carry):
        for b in range(NBUF):
            j = NBUF * i + b
            gather_wait(b)
            pltpu.async_copy(rows[b], acc.at[dst_v.at[j]], ssem[b], add=True)
            nxt = j + 2
            b2 = (b + 2) % NBUF

            @pl.when(nxt < CHUNKS_PER_TILE)
            def _():
                @pl.when(nxt >= NBUF)
                def _():
                    scatter_wait(b2)

                gather(nxt, b2)
        return carry

    lax.fori_loop(0, CHUNKS_PER_TILE // NBUF, chunk_body, 0)
    for b in range(NBUF):
        scatter_wait(b)
    plsc.subcore_barrier()

    @pl.when(s < NUM_TILES - 1)
    def _():
        pltpu.sync_copy(acc.at[pl.ds(s * ZERO_ROWS, ZERO_ROWS)],
                        out_hbm.at[c].at[pl.ds(s * ZERO_ROWS, ZERO_ROWS)])

    @pl.when(s == NUM_TILES - 1)
    def _():
        base = (NUM_TILES - 1) * ZERO_ROWS
        pltpu.sync_copy(acc.at[pl.ds(base, OUT_ROWS_LAST)],
                        out_hbm.at[c].at[pl.ds(base, OUT_ROWS_LAST)])


def _sc_entry(x_lo_hbm, x_hi_hbm, src_hbm, dst_hbm, z_hbm, out_hbm,
              acc, src_v, dst_v, r0, r1, r2, r3,
              g0, g1, g2, g3, s0, s1, s2, s3):
    _sc_kernel(x_lo_hbm, x_hi_hbm, src_hbm, dst_hbm, z_hbm, out_hbm,
               acc, src_v, dst_v, (r0, r1, r2, r3),
               (g0, g1, g2, g3), (s0, s1, s2, s3))


@jax.jit
def kernel(x, edge_neighbors):
    en = edge_neighbors.astype(jnp.int32)
    pad = E_PAD - N_EDGES
    src = jnp.concatenate([en[1], jnp.zeros((pad,), jnp.int32)])
    dst = jnp.concatenate([en[0], jnp.full((pad,), N_NODES, jnp.int32)])
    src = src.reshape(NUM_TILES, CHUNKS_PER_TILE, CHUNK)
    dst = dst.reshape(NUM_TILES, CHUNKS_PER_TILE, CHUNK)
    x_lo = x[:, :D_HALF]
    x_hi = x[:, D_HALF:]
    zeros = jnp.zeros((ROWS_PAD, D_HALF), jnp.float32)

    mesh = plsc.VectorSubcoreMesh(core_axis_name="c", subcore_axis_name="s")
    run = functools.partial(
        pl.kernel,
        mesh=mesh,
        compiler_params=pltpu.CompilerParams(use_tc_tiling_on_sc=False),
        out_type=jax.ShapeDtypeStruct((NUM_CORES, N_NODES, D_HALF), jnp.float32),
        scratch_types=[
            pltpu.VMEM_SHARED((ROWS_PAD, D_HALF), jnp.float32),
            pltpu.VMEM((CHUNKS_PER_TILE, CHUNK), jnp.int32),
            pltpu.VMEM((CHUNKS_PER_TILE, CHUNK), jnp.int32),
            *[pltpu.VMEM((CHUNK, D_HALF), jnp.float32) for _ in range(NBUF)],
            *[pltpu.SemaphoreType.DMA for _ in range(2 * NBUF)],
        ],
    )(_sc_entry)
    out3 = run(x_lo, x_hi, src, dst, zeros)
    return jnp.concatenate([out3[0], out3[1]], axis=1)


# speedup vs baseline: 1.3951x; 1.3951x over previous
import functools

import jax
import jax.numpy as jnp
from jax import lax
from jax.experimental import pallas as pl
from jax.experimental.pallas import tpu as pltpu
from jax.experimental.pallas import tpu_sc as plsc

N_NODES = 10000
N_EDGES = 320000
D_FEAT = 128

NUM_CORES = 2
NUM_TILES = 16
CHUNK = 128
D_HALF = D_FEAT // NUM_CORES

CHUNKS_PER_TILE = 158
E_PAD = CHUNKS_PER_TILE * NUM_TILES * CHUNK
ZERO_ROWS = 632
ROWS_PAD = ZERO_ROWS * NUM_TILES
OUT_ROWS_LAST = N_NODES - ZERO_ROWS * (NUM_TILES - 1)


def _sc_kernel(x_lo_hbm, x_hi_hbm, ed_hbm, z_hbm, out_hbm,
               acc, x_sc, ed0, ed1, rows0, rows1, g0, g1, i0, i1):
    c = lax.axis_index("c")
    s = lax.axis_index("s")
    rows = (rows0, rows1)
    ed = (ed0, ed1)
    gsem = (g0, g1)
    isem = (i0, i1)

    pltpu.sync_copy(z_hbm.at[pl.ds(s * ZERO_ROWS, ZERO_ROWS)],
                    acc.at[pl.ds(s * ZERO_ROWS, ZERO_ROWS)])

    def stage(x_half):
        @pl.when(s < NUM_TILES - 1)
        def _():
            pltpu.sync_copy(x_half.at[pl.ds(s * ZERO_ROWS, ZERO_ROWS)],
                            x_sc.at[pl.ds(s * ZERO_ROWS, ZERO_ROWS)])

        @pl.when(s == NUM_TILES - 1)
        def _():
            base = (NUM_TILES - 1) * ZERO_ROWS
            pltpu.sync_copy(x_half.at[pl.ds(base, OUT_ROWS_LAST)],
                            x_sc.at[pl.ds(base, OUT_ROWS_LAST)])

    @pl.when(c == 0)
    def _():
        stage(x_lo_hbm)

    @pl.when(c == 1)
    def _():
        stage(x_hi_hbm)

    plsc.subcore_barrier()

    def gather(j_ignored, b):
        pltpu.async_copy(x_sc.at[ed[b].at[0]], rows[b], gsem[b])

    def gwait(b):
        pltpu.make_async_copy(x_sc.at[pl.ds(0, CHUNK)], rows[b],
                              gsem[b]).wait()

    def iwait(b):
        pltpu.make_async_copy(ed_hbm.at[s, 0], ed[b], isem[b]).wait()

    pltpu.sync_copy(ed_hbm.at[s, 0], ed0)
    pltpu.async_copy(ed_hbm.at[s, 1], ed1, i1)
    gather(0, 0)

    def chunk_body(i, carry):
        for b in (0, 1):
            j = 2 * i + b
            gwait(b)

            @pl.when(j + 1 < CHUNKS_PER_TILE)
            def _():
                iwait(1 - b)
                gather(j + 1, 1 - b)

            pltpu.sync_copy(rows[b], acc.at[ed[b].at[1]], add=True)

            @pl.when(j + 2 < CHUNKS_PER_TILE)
            def _():
                pltpu.async_copy(ed_hbm.at[s, j + 2], ed[b], isem[b])
        return carry

    lax.fori_loop(0, CHUNKS_PER_TILE // 2, chunk_body, 0)
    plsc.subcore_barrier()

    @pl.when(s < NUM_TILES - 1)
    def _():
        pltpu.sync_copy(acc.at[pl.ds(s * ZERO_ROWS, ZERO_ROWS)],
                        out_hbm.at[c].at[pl.ds(s * ZERO_ROWS, ZERO_ROWS)])

    @pl.when(s == NUM_TILES - 1)
    def _():
        base = (NUM_TILES - 1) * ZERO_ROWS
        pltpu.sync_copy(acc.at[pl.ds(base, OUT_ROWS_LAST)],
                        out_hbm.at[c].at[pl.ds(base, OUT_ROWS_LAST)])


@jax.jit
def kernel(x, edge_neighbors):
    en = edge_neighbors.astype(jnp.int32)
    pad = E_PAD - N_EDGES
    src = jnp.concatenate([en[1], jnp.zeros((pad,), jnp.int32)])
    dst = jnp.concatenate([en[0], jnp.full((pad,), N_NODES, jnp.int32)])
    ed = jnp.concatenate(
        [src.reshape(NUM_TILES, CHUNKS_PER_TILE, 1, CHUNK),
         dst.reshape(NUM_TILES, CHUNKS_PER_TILE, 1, CHUNK)], axis=2)
    x_lo = x[:, :D_HALF]
    x_hi = x[:, D_HALF:]
    zeros = jnp.zeros((ROWS_PAD, D_HALF), jnp.float32)

    mesh = plsc.VectorSubcoreMesh(core_axis_name="c", subcore_axis_name="s")
    run = functools.partial(
        pl.kernel,
        mesh=mesh,
        compiler_params=pltpu.CompilerParams(use_tc_tiling_on_sc=False),
        out_type=jax.ShapeDtypeStruct((NUM_CORES, N_NODES, D_HALF), jnp.float32),
        scratch_types=[
            pltpu.VMEM_SHARED((ROWS_PAD, D_HALF), jnp.float32),
            pltpu.VMEM_SHARED((N_NODES, D_HALF), jnp.float32),
            pltpu.VMEM((2, CHUNK), jnp.int32),
            pltpu.VMEM((2, CHUNK), jnp.int32),
            pltpu.VMEM((CHUNK, D_HALF), jnp.float32),
            pltpu.VMEM((CHUNK, D_HALF), jnp.float32),
            pltpu.SemaphoreType.DMA,
            pltpu.SemaphoreType.DMA,
            pltpu.SemaphoreType.DMA,
            pltpu.SemaphoreType.DMA,
        ],
    )(_sc_kernel)
    out3 = run(x_lo, x_hi, ed, zeros)
    return jnp.concatenate([out3[0], out3[1]], axis=1)


# speedup vs baseline: 1.6918x; 1.2127x over previous
import functools

import jax
import jax.numpy as jnp
from jax import lax
from jax.experimental import pallas as pl
from jax.experimental.pallas import tpu as pltpu
from jax.experimental.pallas import tpu_sc as plsc

N_NODES = 10000
N_EDGES = 320000
D_FEAT = 128

NUM_CORES = 2
NUM_TILES = 16
CHUNK = 128
D_HALF = D_FEAT // NUM_CORES

CHUNKS_PER_TILE = 156
E_MAIN = CHUNKS_PER_TILE * NUM_TILES * CHUNK
TAIL_CHUNKS = (N_EDGES - E_MAIN) // CHUNK
ZERO_ROWS = 632
ROWS_PAD = ZERO_ROWS * NUM_TILES
OUT_ROWS_LAST = N_NODES - ZERO_ROWS * (NUM_TILES - 1)


def _sc_kernel(x_hbm, en_hbm, z_hbm, out_hbm,
               acc, x_sc, src0, src1, dst0, dst1, rows0, rows1,
               g0, g1, is0, is1, id0, id1):
    c = lax.axis_index("c")
    s = lax.axis_index("s")
    rows = (rows0, rows1)
    srcv = (src0, src1)
    dstv = (dst0, dst1)
    gsem = (g0, g1)
    issem = (is0, is1)
    idsem = (id0, id1)

    pltpu.sync_copy(z_hbm.at[pl.ds(s * ZERO_ROWS, ZERO_ROWS)],
                    acc.at[pl.ds(s * ZERO_ROWS, ZERO_ROWS)])

    @pl.when(s < NUM_TILES - 1)
    def _():
        pltpu.sync_copy(
            x_hbm.at[pl.ds(s * ZERO_ROWS, ZERO_ROWS), pl.ds(c * D_HALF, D_HALF)],
            x_sc.at[pl.ds(s * ZERO_ROWS, ZERO_ROWS)])

    @pl.when(s == NUM_TILES - 1)
    def _():
        base = (NUM_TILES - 1) * ZERO_ROWS
        pltpu.sync_copy(
            x_hbm.at[pl.ds(base, OUT_ROWS_LAST), pl.ds(c * D_HALF, D_HALF)],
            x_sc.at[pl.ds(base, OUT_ROWS_LAST)])

    plsc.subcore_barrier()

    def gather(b):
        pltpu.async_copy(x_sc.at[srcv[b]], rows[b], gsem[b])

    def gwait(b):
        pltpu.make_async_copy(x_sc.at[pl.ds(0, CHUNK)], rows[b],
                              gsem[b]).wait()

    def load_idx(e0, b):
        pltpu.async_copy(en_hbm.at[1, pl.ds(e0, CHUNK)], srcv[b], issem[b])
        pltpu.async_copy(en_hbm.at[0, pl.ds(e0, CHUNK)], dstv[b], idsem[b])

    def iwait(b):
        pltpu.make_async_copy(en_hbm.at[1, pl.ds(0, CHUNK)], srcv[b],
                              issem[b]).wait()
        pltpu.make_async_copy(en_hbm.at[0, pl.ds(0, CHUNK)], dstv[b],
                              idsem[b]).wait()

    @pl.when(s < TAIL_CHUNKS)
    def _():
        load_idx(E_MAIN + s * CHUNK, 0)
        iwait(0)
        gather(0)
        gwait(0)
        pltpu.sync_copy(rows[0], acc.at[dstv[0]], add=True)

    base_e = s * (CHUNKS_PER_TILE * CHUNK)
    load_idx(base_e, 0)
    load_idx(base_e + CHUNK, 1)
    iwait(0)
    gather(0)

    def chunk_body(i, carry):
        for b in (0, 1):
            j = 2 * i + b
            gwait(b)

            @pl.when(j + 1 < CHUNKS_PER_TILE)
            def _():
                iwait(1 - b)
                gather(1 - b)

            pltpu.sync_copy(rows[b], acc.at[dstv[b]], add=True)

            @pl.when(j + 2 < CHUNKS_PER_TILE)
            def _():
                load_idx(base_e + (j + 2) * CHUNK, b)
        return carry

    lax.fori_loop(0, CHUNKS_PER_TILE // 2, chunk_body, 0)
    plsc.subcore_barrier()

    @pl.when(s < NUM_TILES - 1)
    def _():
        pltpu.sync_copy(
            acc.at[pl.ds(s * ZERO_ROWS, ZERO_ROWS)],
            out_hbm.at[pl.ds(s * ZERO_ROWS, ZERO_ROWS), pl.ds(c * D_HALF, D_HALF)])

    @pl.when(s == NUM_TILES - 1)
    def _():
        base = (NUM_TILES - 1) * ZERO_ROWS
        pltpu.sync_copy(
            acc.at[pl.ds(base, OUT_ROWS_LAST)],
            out_hbm.at[pl.ds(base, OUT_ROWS_LAST), pl.ds(c * D_HALF, D_HALF)])


@jax.jit
def kernel(x, edge_neighbors):
    en = edge_neighbors.astype(jnp.int32)
    zeros = jnp.zeros((ROWS_PAD, D_HALF), jnp.float32)

    mesh = plsc.VectorSubcoreMesh(core_axis_name="c", subcore_axis_name="s")
    run = functools.partial(
        pl.kernel,
        mesh=mesh,
        compiler_params=pltpu.CompilerParams(use_tc_tiling_on_sc=False),
        out_type=jax.ShapeDtypeStruct((N_NODES, D_FEAT), jnp.float32),
        scratch_types=[
            pltpu.VMEM_SHARED((ROWS_PAD, D_HALF), jnp.float32),
            pltpu.VMEM_SHARED((N_NODES, D_HALF), jnp.float32),
            pltpu.VMEM((CHUNK,), jnp.int32),
            pltpu.VMEM((CHUNK,), jnp.int32),
            pltpu.VMEM((CHUNK,), jnp.int32),
            pltpu.VMEM((CHUNK,), jnp.int32),
            pltpu.VMEM((CHUNK, D_HALF), jnp.float32),
            pltpu.VMEM((CHUNK, D_HALF), jnp.float32),
            pltpu.SemaphoreType.DMA,
            pltpu.SemaphoreType.DMA,
            pltpu.SemaphoreType.DMA,
            pltpu.SemaphoreType.DMA,
            pltpu.SemaphoreType.DMA,
            pltpu.SemaphoreType.DMA,
        ],
    )(_sc_kernel)
    return run(x, en, zeros)


# speedup vs baseline: 2.0647x; 1.2204x over previous
import functools

import jax
import jax.numpy as jnp
from jax import lax
from jax.experimental import pallas as pl
from jax.experimental.pallas import tpu as pltpu
from jax.experimental.pallas import tpu_sc as plsc

N_NODES = 10000
N_EDGES = 320000
D_FEAT = 128

NUM_CORES = 2
NUM_TILES = 16
CHUNK = 128
D_HALF = D_FEAT // NUM_CORES
NBUF = 4

CHUNKS_PER_TILE = 156
E_MAIN = CHUNKS_PER_TILE * NUM_TILES * CHUNK
TAIL_CHUNKS = (N_EDGES - E_MAIN) // CHUNK
ZERO_ROWS = 632
ROWS_PAD = ZERO_ROWS * NUM_TILES
OUT_ROWS_LAST = N_NODES - ZERO_ROWS * (NUM_TILES - 1)


def _sc_kernel(x_lo_hbm, x_hi_hbm, en_hbm, z_hbm, out_hbm,
               acc, x_sc, srcv, dstv, rows, gsem, issem, idsem):
    c = lax.axis_index("c")
    s = lax.axis_index("s")

    pltpu.sync_copy(z_hbm.at[pl.ds(s * ZERO_ROWS, ZERO_ROWS)],
                    acc.at[pl.ds(s * ZERO_ROWS, ZERO_ROWS)])

    def stage(x_half):
        @pl.when(s < NUM_TILES - 1)
        def _():
            pltpu.sync_copy(x_half.at[pl.ds(s * ZERO_ROWS, ZERO_ROWS)],
                            x_sc.at[pl.ds(s * ZERO_ROWS, ZERO_ROWS)])

        @pl.when(s == NUM_TILES - 1)
        def _():
            base = (NUM_TILES - 1) * ZERO_ROWS
            pltpu.sync_copy(x_half.at[pl.ds(base, OUT_ROWS_LAST)],
                            x_sc.at[pl.ds(base, OUT_ROWS_LAST)])

    @pl.when(c == 0)
    def _():
        stage(x_lo_hbm)

    @pl.when(c == 1)
    def _():
        stage(x_hi_hbm)

    plsc.subcore_barrier()

    def gather_spmem(b):
        pltpu.async_copy(x_sc.at[srcv[b]], rows[b], gsem[b])

    def gather_hbm(b):
        @pl.when(c == 0)
        def _():
            pltpu.async_copy(x_lo_hbm.at[srcv[b]], rows[b], gsem[b])

        @pl.when(c == 1)
        def _():
            pltpu.async_copy(x_hi_hbm.at[srcv[b]], rows[b], gsem[b])

    def gather(b):
        if b % 2 == 0:
            gather_spmem(b)
        else:
            gather_hbm(b)

    def gwait(b):
        if b % 2 == 0:
            pltpu.make_async_copy(x_sc.at[pl.ds(0, CHUNK)], rows[b],
                                  gsem[b]).wait()
        else:
            pltpu.make_async_copy(x_lo_hbm.at[pl.ds(0, CHUNK)], rows[b],
                                  gsem[b]).wait()

    def load_idx(e0, b):
        pltpu.async_copy(en_hbm.at[1, pl.ds(e0, CHUNK)], srcv[b], issem[b])
        pltpu.async_copy(en_hbm.at[0, pl.ds(e0, CHUNK)], dstv[b], idsem[b])

    def iwait(b):
        pltpu.make_async_copy(en_hbm.at[1, pl.ds(0, CHUNK)], srcv[b],
                              issem[b]).wait()
        pltpu.make_async_copy(en_hbm.at[0, pl.ds(0, CHUNK)], dstv[b],
                              idsem[b]).wait()

    @pl.when(s < TAIL_CHUNKS)
    def _():
        load_idx(E_MAIN + s * CHUNK, 0)
        iwait(0)
        gather_spmem(0)
        gwait(0)
        pltpu.sync_copy(rows[0], acc.at[dstv[0]], add=True)

    base_e = s * (CHUNKS_PER_TILE * CHUNK)
    for b in range(NBUF):
        load_idx(base_e + b * CHUNK, b)
    iwait(0)
    gather(0)
    iwait(1)
    gather(1)

    def chunk_body(i, carry):
        for b in range(NBUF):
            j = NBUF * i + b
            bn = (b + 2) % NBUF
            gwait(b)

            @pl.when(j + 2 < CHUNKS_PER_TILE)
            def _():
                iwait(bn)
                gather(bn)

            pltpu.sync_copy(rows[b], acc.at[dstv[b]], add=True)

            @pl.when(j + NBUF < CHUNKS_PER_TILE)
            def _():
                load_idx(base_e + (j + NBUF) * CHUNK, b)
        return carry

    lax.fori_loop(0, CHUNKS_PER_TILE // NBUF, chunk_body, 0)
    plsc.subcore_barrier()

    @pl.when(s < NUM_TILES - 1)
    def _():
        pltpu.sync_copy(
            acc.at[pl.ds(s * ZERO_ROWS, ZERO_ROWS)],
            out_hbm.at[pl.ds(s * ZERO_ROWS, ZERO_ROWS), pl.ds(c * D_HALF, D_HALF)])

    @pl.when(s == NUM_TILES - 1)
    def _():
        base = (NUM_TILES - 1) * ZERO_ROWS
        pltpu.sync_copy(
            acc.at[pl.ds(base, OUT_ROWS_LAST)],
            out_hbm.at[pl.ds(base, OUT_ROWS_LAST), pl.ds(c * D_HALF, D_HALF)])


def _sc_entry(x_lo_hbm, x_hi_hbm, en_hbm, z_hbm, out_hbm, acc, x_sc,
              s0, s1, s2, s3, d0, d1, d2, d3, r0, r1, r2, r3,
              g0, g1, g2, g3, a0, a1, a2, a3, b0, b1, b2, b3):
    _sc_kernel(x_lo_hbm, x_hi_hbm, en_hbm, z_hbm, out_hbm, acc, x_sc,
               (s0, s1, s2, s3), (d0, d1, d2, d3), (r0, r1, r2, r3),
               (g0, g1, g2, g3), (a0, a1, a2, a3), (b0, b1, b2, b3))


@jax.jit
def kernel(x, edge_neighbors):
    en = edge_neighbors.astype(jnp.int32)
    x_lo = x[:, :D_HALF]
    x_hi = x[:, D_HALF:]
    zeros = jnp.zeros((ROWS_PAD, D_HALF), jnp.float32)

    mesh = plsc.VectorSubcoreMesh(core_axis_name="c", subcore_axis_name="s")
    run = functools.partial(
        pl.kernel,
        mesh=mesh,
        compiler_params=pltpu.CompilerParams(use_tc_tiling_on_sc=False),
        out_type=jax.ShapeDtypeStruct((N_NODES, D_FEAT), jnp.float32),
        scratch_types=[
            pltpu.VMEM_SHARED((ROWS_PAD, D_HALF), jnp.float32),
            pltpu.VMEM_SHARED((N_NODES, D_HALF), jnp.float32),
            *[pltpu.VMEM((CHUNK,), jnp.int32) for _ in range(NBUF)],
            *[pltpu.VMEM((CHUNK,), jnp.int32) for _ in range(NBUF)],
            *[pltpu.VMEM((CHUNK, D_HALF), jnp.float32) for _ in range(NBUF)],
            *[pltpu.SemaphoreType.DMA for _ in range(3 * NBUF)],
        ],
    )(_sc_entry)
    return run(x_lo, x_hi, en, zeros)


# speedup vs baseline: 2.0874x; 1.0110x over previous
import functools

import jax
import jax.numpy as jnp
from jax import lax
from jax.experimental import pallas as pl
from jax.experimental.pallas import tpu as pltpu
from jax.experimental.pallas import tpu_sc as plsc

N_NODES = 10000
N_EDGES = 320000
D_FEAT = 128

NUM_CORES = 2
NUM_TILES = 16
CHUNK = 128
D_HALF = D_FEAT // NUM_CORES
NROWS = 4
NIDX = 8

CHUNKS_PER_TILE = 156
LOOP_CHUNKS = (CHUNKS_PER_TILE // NIDX) * NIDX
E_MAIN = CHUNKS_PER_TILE * NUM_TILES * CHUNK
TAIL_CHUNKS = (N_EDGES - E_MAIN) // CHUNK
ZERO_ROWS = 632
ROWS_PAD = ZERO_ROWS * NUM_TILES
OUT_ROWS_LAST = N_NODES - ZERO_ROWS * (NUM_TILES - 1)


def _sc_kernel(x_lo_hbm, x_hi_hbm, en_hbm, z_hbm, out_hbm,
               acc, x_sc, srcv, dstv, rows, gsem, ssem, issem, idsem):
    c = lax.axis_index("c")
    s = lax.axis_index("s")

    pltpu.sync_copy(z_hbm.at[pl.ds(s * ZERO_ROWS, ZERO_ROWS)],
                    acc.at[pl.ds(s * ZERO_ROWS, ZERO_ROWS)])

    def stage(x_half):
        @pl.when(s < NUM_TILES - 1)
        def _():
            pltpu.sync_copy(x_half.at[pl.ds(s * ZERO_ROWS, ZERO_ROWS)],
                            x_sc.at[pl.ds(s * ZERO_ROWS, ZERO_ROWS)])

        @pl.when(s == NUM_TILES - 1)
        def _():
            base = (NUM_TILES - 1) * ZERO_ROWS
            pltpu.sync_copy(x_half.at[pl.ds(base, OUT_ROWS_LAST)],
                            x_sc.at[pl.ds(base, OUT_ROWS_LAST)])

    @pl.when(c == 0)
    def _():
        stage(x_lo_hbm)

    @pl.when(c == 1)
    def _():
        stage(x_hi_hbm)

    plsc.subcore_barrier()

    def gather(q, r):
        if q % 2 == 0:
            pltpu.async_copy(x_sc.at[srcv[q]], rows[r], gsem[r])
        else:
            @pl.when(c == 0)
            def _():
                pltpu.async_copy(x_lo_hbm.at[srcv[q]], rows[r], gsem[r])

            @pl.when(c == 1)
            def _():
                pltpu.async_copy(x_hi_hbm.at[srcv[q]], rows[r], gsem[r])

    def gwait(r):
        if r % 2 == 0:
            pltpu.make_async_copy(x_sc.at[pl.ds(0, CHUNK)], rows[r],
                                  gsem[r]).wait()
        else:
            pltpu.make_async_copy(x_lo_hbm.at[pl.ds(0, CHUNK)], rows[r],
                                  gsem[r]).wait()

    def swait(r):
        pltpu.make_async_copy(rows[r], acc.at[pl.ds(0, CHUNK)],
                              ssem[r]).wait()

    def load_idx(e0, q):
        pltpu.async_copy(en_hbm.at[1, pl.ds(e0, CHUNK)], srcv[q], issem[q])
        pltpu.async_copy(en_hbm.at[0, pl.ds(e0, CHUNK)], dstv[q], idsem[q])

    def iwait(q):
        pltpu.make_async_copy(en_hbm.at[1, pl.ds(0, CHUNK)], srcv[q],
                              issem[q]).wait()
        pltpu.make_async_copy(en_hbm.at[0, pl.ds(0, CHUNK)], dstv[q],
                              idsem[q]).wait()

    @pl.when(s < TAIL_CHUNKS)
    def _():
        load_idx(E_MAIN + s * CHUNK, 0)
        iwait(0)
        pltpu.async_copy(x_sc.at[srcv[0]], rows[0], gsem[0])
        pltpu.make_async_copy(x_sc.at[pl.ds(0, CHUNK)], rows[0],
                              gsem[0]).wait()
        pltpu.sync_copy(rows[0], acc.at[dstv[0]], add=True)

    base_e = s * (CHUNKS_PER_TILE * CHUNK)
    for q in range(6):
        load_idx(base_e + q * CHUNK, q)
    iwait(0)
    gather(0, 0)
    iwait(1)
    gather(1, 1)

    def step(j, b, guard_swait, in_loop):
        q, r = b % NIDX, b % NROWS
        q2, r2 = (b + 2) % NIDX, (b + 2) % NROWS
        gwait(r)
        pltpu.async_copy(rows[r], acc.at[dstv[q]], ssem[r], add=True)

        def prefetch_gather():
            iwait(q2)
            if guard_swait:
                @pl.when(j >= 2)
                def _():
                    swait(r2)
            else:
                swait(r2)
            gather(q2, r2)

        if in_loop:
            prefetch_gather()

            @pl.when(j + 6 < CHUNKS_PER_TILE)
            def _():
                load_idx(base_e + (j + 6) * CHUNK, (b + 6) % NIDX)
        else:
            if j + 2 < CHUNKS_PER_TILE:
                prefetch_gather()

    def chunk_body(i, carry):
        for b in range(NIDX):
            step(NIDX * i + b, b, guard_swait=(b < 2), in_loop=True)
        return carry

    lax.fori_loop(0, LOOP_CHUNKS // NIDX, chunk_body, 0)

    for j in range(LOOP_CHUNKS, CHUNKS_PER_TILE):
        step(j, j, guard_swait=False, in_loop=False)
    for r in range(NROWS):
        swait(r)
    plsc.subcore_barrier()

    @pl.when(s < NUM_TILES - 1)
    def _():
        pltpu.sync_copy(
            acc.at[pl.ds(s * ZERO_ROWS, ZERO_ROWS)],
            out_hbm.at[pl.ds(s * ZERO_ROWS, ZERO_ROWS), pl.ds(c * D_HALF, D_HALF)])

    @pl.when(s == NUM_TILES - 1)
    def _():
        base = (NUM_TILES - 1) * ZERO_ROWS
        pltpu.sync_copy(
            acc.at[pl.ds(base, OUT_ROWS_LAST)],
            out_hbm.at[pl.ds(base, OUT_ROWS_LAST), pl.ds(c * D_HALF, D_HALF)])


def _sc_entry(x_lo_hbm, x_hi_hbm, en_hbm, z_hbm, out_hbm, acc, x_sc,
              sv0, sv1, sv2, sv3, sv4, sv5, sv6, sv7,
              dv0, dv1, dv2, dv3, dv4, dv5, dv6, dv7,
              r0, r1, r2, r3,
              g0, g1, g2, g3, ss0, ss1, ss2, ss3,
              ia0, ia1, ia2, ia3, ia4, ia5, ia6, ia7,
              ib0, ib1, ib2, ib3, ib4, ib5, ib6, ib7):
    _sc_kernel(x_lo_hbm, x_hi_hbm, en_hbm, z_hbm, out_hbm, acc, x_sc,
               (sv0, sv1, sv2, sv3, sv4, sv5, sv6, sv7),
               (dv0, dv1, dv2, dv3, dv4, dv5, dv6, dv7),
               (r0, r1, r2, r3),
               (g0, g1, g2, g3), (ss0, ss1, ss2, ss3),
               (ia0, ia1, ia2, ia3, ia4, ia5, ia6, ia7),
               (ib0, ib1, ib2, ib3, ib4, ib5, ib6, ib7))


@jax.jit
def kernel(x, edge_neighbors):
    en = edge_neighbors.astype(jnp.int32)
    x_lo = x[:, :D_HALF]
    x_hi = x[:, D_HALF:]
    zeros = jnp.zeros((ROWS_PAD, D_HALF), jnp.float32)

    mesh = plsc.VectorSubcoreMesh(core_axis_name="c", subcore_axis_name="s")
    run = functools.partial(
        pl.kernel,
        mesh=mesh,
        compiler_params=pltpu.CompilerParams(use_tc_tiling_on_sc=False),
        out_type=jax.ShapeDtypeStruct((N_NODES, D_FEAT), jnp.float32),
        scratch_types=[
            pltpu.VMEM_SHARED((ROWS_PAD, D_HALF), jnp.float32),
            pltpu.VMEM_SHARED((N_NODES, D_HALF), jnp.float32),
            *[pltpu.VMEM((CHUNK,), jnp.int32) for _ in range(NIDX)],
            *[pltpu.VMEM((CHUNK,), jnp.int32) for _ in range(NIDX)],
            *[pltpu.VMEM((CHUNK, D_HALF), jnp.float32) for _ in range(NROWS)],
            *[pltpu.SemaphoreType.DMA for _ in range(2 * NROWS)],
            *[pltpu.SemaphoreType.DMA for _ in range(2 * NIDX)],
        ],
    )(_sc_entry)
    return run(x_lo, x_hi, en, zeros)


# speedup vs baseline: 2.0903x; 1.0014x over previous
import functools

import jax
import jax.numpy as jnp
from jax import lax
from jax.experimental import pallas as pl
from jax.experimental.pallas import tpu as pltpu
from jax.experimental.pallas import tpu_sc as plsc

N_NODES = 10000
N_EDGES = 320000
D_FEAT = 128

NUM_CORES = 2
NUM_TILES = 16
CHUNK = 128
D_HALF = D_FEAT // NUM_CORES
NROWS = 4
NIDX = 8

CHUNKS_PER_TILE = 156
N_CHUNKS = N_EDGES // CHUNK
MAIN_CHUNKS = CHUNKS_PER_TILE * NUM_TILES
TAIL_CHUNKS = N_CHUNKS - MAIN_CHUNKS
ZERO_ROWS = 632
ROWS_PAD = ZERO_ROWS * NUM_TILES
OUT_ROWS_LAST = N_NODES - ZERO_ROWS * (NUM_TILES - 1)


def _sc_kernel(x_lo_hbm, x_hi_hbm, en_hbm, z_hbm, out_hbm,
               acc, x_sc, edv, rows, gsem, ssem, isem):
    c = lax.axis_index("c")
    s = lax.axis_index("s")

    pltpu.sync_copy(z_hbm.at[pl.ds(s * ZERO_ROWS, ZERO_ROWS)],
                    acc.at[pl.ds(s * ZERO_ROWS, ZERO_ROWS)])

    def stage(x_half):
        @pl.when(s < NUM_TILES - 1)
        def _():
            pltpu.sync_copy(x_half.at[pl.ds(s * ZERO_ROWS, ZERO_ROWS)],
                            x_sc.at[pl.ds(s * ZERO_ROWS, ZERO_ROWS)])

        @pl.when(s == NUM_TILES - 1)
        def _():
            base = (NUM_TILES - 1) * ZERO_ROWS
            pltpu.sync_copy(x_half.at[pl.ds(base, OUT_ROWS_LAST)],
                            x_sc.at[pl.ds(base, OUT_ROWS_LAST)])

    @pl.when(c == 0)
    def _():
        stage(x_lo_hbm)

    @pl.when(c == 1)
    def _():
        stage(x_hi_hbm)

    plsc.subcore_barrier()

    def gather(q, r):
        if q % 2 == 0:
            pltpu.async_copy(x_sc.at[edv[q].at[1]], rows[r], gsem[r])
        else:
            @pl.when(c == 0)
            def _():
                pltpu.async_copy(x_lo_hbm.at[edv[q].at[1]], rows[r], gsem[r])

            @pl.when(c == 1)
            def _():
                pltpu.async_copy(x_hi_hbm.at[edv[q].at[1]], rows[r], gsem[r])

    def gwait(r):
        if r % 2 == 0:
            pltpu.make_async_copy(x_sc.at[pl.ds(0, CHUNK)], rows[r],
                                  gsem[r]).wait()
        else:
            pltpu.make_async_copy(x_lo_hbm.at[pl.ds(0, CHUNK)], rows[r],
                                  gsem[r]).wait()

    def swait(r):
        pltpu.make_async_copy(rows[r], acc.at[pl.ds(0, CHUNK)],
                              ssem[r]).wait()

    def load_idx(g, q):
        pltpu.async_copy(en_hbm.at[:, g], edv[q], isem[q])

    def iwait(q):
        pltpu.make_async_copy(en_hbm.at[:, 0], edv[q], isem[q]).wait()

    @pl.when(s < TAIL_CHUNKS)
    def _():
        load_idx(MAIN_CHUNKS + s, 0)
        iwait(0)
        pltpu.async_copy(x_sc.at[edv[0].at[1]], rows[0], gsem[0])
        pltpu.make_async_copy(x_sc.at[pl.ds(0, CHUNK)], rows[0],
                              gsem[0]).wait()
        pltpu.sync_copy(rows[0], acc.at[edv[0].at[0]], add=True)

    base_ch = s * CHUNKS_PER_TILE
    for q in range(6):
        load_idx(base_ch + q, q)
    iwait(0)
    gather(0, 0)
    iwait(1)
    gather(1, 1)

    def step(j, b, do_swait, do_prefetch, do_load):
        q, r = b % NIDX, b % NROWS
        q2, r2 = (b + 2) % NIDX, (b + 2) % NROWS
        gwait(r)
        pltpu.async_copy(rows[r], acc.at[edv[q].at[0]], ssem[r], add=True)
        if do_prefetch:
            iwait(q2)
            if do_swait:
                swait(r2)
            gather(q2, r2)
        if do_load:
            load_idx(base_ch + j + 6, (b + 6) % NIDX)

    for j in range(NIDX):
        step(j, j, do_swait=(j >= 2), do_prefetch=True, do_load=True)

    def chunk_body(i, carry):
        for b in range(NIDX):
            step(NIDX * i + b, b, do_swait=True, do_prefetch=True,
                 do_load=True)
        return carry

    lax.fori_loop(1, CHUNKS_PER_TILE // NIDX, chunk_body, 0)

    for j in range(CHUNKS_PER_TILE - NROWS, CHUNKS_PER_TILE):
        step(j, j, do_swait=True,
             do_prefetch=(j + 2 < CHUNKS_PER_TILE), do_load=False)
    for r in range(NROWS):
        swait(r)
    iwait(4)
    iwait(5)
    plsc.subcore_barrier()

    @pl.when(s < NUM_TILES - 1)
    def _():
        pltpu.sync_copy(
            acc.at[pl.ds(s * ZERO_ROWS, ZERO_ROWS)],
            out_hbm.at[pl.ds(s * ZERO_ROWS, ZERO_ROWS), pl.ds(c * D_HALF, D_HALF)])

    @pl.when(s == NUM_TILES - 1)
    def _():
        base = (NUM_TILES - 1) * ZERO_ROWS
        pltpu.sync_copy(
            acc.at[pl.ds(base, OUT_ROWS_LAST)],
            out_hbm.at[pl.ds(base, OUT_ROWS_LAST), pl.ds(c * D_HALF, D_HALF)])


def _sc_entry(x_lo_hbm, x_hi_hbm, en_hbm, z_hbm, out_hbm, acc, x_sc,
              e0, e1, e2, e3, e4, e5, e6, e7,
              r0, r1, r2, r3,
              g0, g1, g2, g3, ss0, ss1, ss2, ss3,
              i0, i1, i2, i3, i4, i5, i6, i7):
    _sc_kernel(x_lo_hbm, x_hi_hbm, en_hbm, z_hbm, out_hbm, acc, x_sc,
               (e0, e1, e2, e3, e4, e5, e6, e7),
               (r0, r1, r2, r3),
               (g0, g1, g2, g3), (ss0, ss1, ss2, ss3),
               (i0, i1, i2, i3, i4, i5, i6, i7))


@jax.jit
def kernel(x, edge_neighbors):
    en = edge_neighbors.astype(jnp.int32).reshape(2, N_CHUNKS, CHUNK)
    x_lo = x[:, :D_HALF]
    x_hi = x[:, D_HALF:]
    zeros = jnp.zeros((ROWS_PAD, D_HALF), jnp.float32)

    mesh = plsc.VectorSubcoreMesh(core_axis_name="c", subcore_axis_name="s")
    run = functools.partial(
        pl.kernel,
        mesh=mesh,
        compiler_params=pltpu.CompilerParams(use_tc_tiling_on_sc=False),
        out_type=jax.ShapeDtypeStruct((N_NODES, D_FEAT), jnp.float32),
        scratch_types=[
            pltpu.VMEM_SHARED((ROWS_PAD, D_HALF), jnp.float32),
            pltpu.VMEM_SHARED((N_NODES, D_HALF), jnp.float32),
            *[pltpu.VMEM((2, CHUNK), jnp.int32) for _ in range(NIDX)],
            *[pltpu.VMEM((CHUNK, D_HALF), jnp.float32) for _ in range(NROWS)],
            *[pltpu.SemaphoreType.DMA for _ in range(2 * NROWS)],
            *[pltpu.SemaphoreType.DMA for _ in range(NIDX)],
        ],
    )(_sc_entry)
    return run(x_lo, x_hi, en, zeros)
